# baseline (speedup 1.0000x reference)
import jax
import jax.numpy as jnp
from jax.experimental import pallas as pl


def _min1_kernel(x_ref, o_ref):
    o_ref[...] = jnp.minimum(x_ref[...], 1.0)


def kernel(x):
    b, m, n = x.shape
    x2 = x.reshape(b * m, n)
    rows = b * m
    block_rows = 1024
    out = pl.pallas_call(
        _min1_kernel,
        out_shape=jax.ShapeDtypeStruct((rows, n), x.dtype),
        grid=(rows // block_rows,),
        in_specs=[pl.BlockSpec((block_rows, n), lambda i: (i, 0))],
        out_specs=pl.BlockSpec((block_rows, n), lambda i: (i, 0)),
    )(x2)
    return out.reshape(b, m, n)
